# Initial kernel scaffold; baseline (speedup 1.0000x reference)
import jax
import jax.numpy as jnp
from jax.experimental import pallas as pl


def kernel(x, W_enc, W_dec, b_dec):
    raise NotImplementedError("write your pallas kernel here")


# speedup vs baseline: 10.0146x; 10.0146x over previous
import jax
import jax.numpy as jnp
from jax.experimental import pallas as pl

_D = 768
_F = 12288
_K = 32
_NP = 3200
_TB_MM = 800
_FB = 512
_TB_TK = 160


def _enc_body(x_ref, w_ref, b_ref, o_ref):
    xc = x_ref[...] - b_ref[...]
    o_ref[...] = jnp.maximum(
        jnp.dot(xc, w_ref[...], preferred_element_type=jnp.float32), 0.0)


def _topk_body(a_ref, f_ref):
    a = a_ref[...]
    iota = jax.lax.broadcasted_iota(jnp.int32, a.shape, 1)

    def step(_, v):
        m = jnp.max(v, axis=1, keepdims=True)
        am = jnp.min(jnp.where(v == m, iota, _F), axis=1, keepdims=True)
        return jnp.where(iota == am, -jnp.inf, v)

    v = jax.lax.fori_loop(0, _K, step, a)
    f_ref[...] = jnp.where(v == -jnp.inf, a, 0.0)


def _dec_body(f_ref, w_ref, b_ref, o_ref):
    k = pl.program_id(1)

    @pl.when(k == 0)
    def _():
        o_ref[...] = jnp.zeros_like(o_ref)

    o_ref[...] += jnp.dot(f_ref[...], w_ref[...],
                          preferred_element_type=jnp.float32)

    @pl.when(k == pl.num_programs(1) - 1)
    def _():
        o_ref[...] += b_ref[...]


def kernel(x, W_enc, W_dec, b_dec):
    B, S, D = x.shape
    N = B * S
    xf = jnp.pad(x.reshape(N, D), ((0, _NP - N), (0, 0)))
    b2 = b_dec.reshape(1, D)

    acts = pl.pallas_call(
        _enc_body,
        grid=(_NP // _TB_MM, _F // _FB),
        in_specs=[
            pl.BlockSpec((_TB_MM, D), lambda t, f: (t, 0)),
            pl.BlockSpec((D, _FB), lambda t, f: (0, f)),
            pl.BlockSpec((1, D), lambda t, f: (0, 0)),
        ],
        out_specs=pl.BlockSpec((_TB_MM, _FB), lambda t, f: (t, f)),
        out_shape=jax.ShapeDtypeStruct((_NP, _F), jnp.float32),
    )(xf, W_enc, b2)

    feats = pl.pallas_call(
        _topk_body,
        grid=(_NP // _TB_TK,),
        in_specs=[pl.BlockSpec((_TB_TK, _F), lambda t: (t, 0))],
        out_specs=pl.BlockSpec((_TB_TK, _F), lambda t: (t, 0)),
        out_shape=jax.ShapeDtypeStruct((_NP, _F), jnp.float32),
    )(acts)

    xhat = pl.pallas_call(
        _dec_body,
        grid=(_NP // _TB_MM, _F // _FB),
        in_specs=[
            pl.BlockSpec((_TB_MM, _FB), lambda t, k: (t, k)),
            pl.BlockSpec((_FB, D), lambda t, k: (k, 0)),
            pl.BlockSpec((1, D), lambda t, k: (0, 0)),
        ],
        out_specs=pl.BlockSpec((_TB_MM, D), lambda t, k: (t, 0)),
        out_shape=jax.ShapeDtypeStruct((_NP, D), jnp.float32),
    )(feats, W_dec, b2)

    return xhat[:N].reshape(B, S, D)


# speedup vs baseline: 16.8119x; 1.6787x over previous
import functools

import jax
import jax.numpy as jnp
from jax import lax
from jax.experimental import pallas as pl
from jax.experimental.pallas import tpu as pltpu
from jax.experimental.pallas import tpu_sc as plsc

_D = 768
_F = 12288
_K = 32
_CH = 128
_NCH = _F // _CH
_NP = 3328
_TB_MM = 832
_FB = 1024
_TB_SEL = 208


def _enc_body(x_ref, w_ref, b_ref, o_ref):
    xc = x_ref[...] - b_ref[...]
    a = jnp.maximum(
        jnp.dot(xc, w_ref[...], preferred_element_type=jnp.float32), 0.0)
    o_ref[...] = a.reshape(a.shape[0], _FB // _CH, _CH)


def _cmax_body(a_ref, o_ref):
    o_ref[...] = jnp.max(a_ref[...], axis=2)


def _chunksel_body(cm_ref, ids_ref, th_ref):
    w = cm_ref[...]
    iota = lax.broadcasted_iota(jnp.int32, w.shape, 1)
    tok0 = pl.program_id(0) * _TB_SEL
    rowbase = (tok0 + lax.broadcasted_iota(jnp.int32, (w.shape[0], 1), 0)) * _NCH
    for j in range(_K):
        m = jnp.max(w, axis=1, keepdims=True)
        am = jnp.min(jnp.where(w == m, iota, _NCH), axis=1, keepdims=True)
        ids_ref[:, j:j + 1] = am + rowbase
        th_ref[...] = m
        w = jnp.where(iota == am, -jnp.inf, w)


def _sc_select_decode(acts_flat, ids, theta, W_dec, b_dec2):
    info = plsc.get_sparse_core_info()
    nw = info.num_cores * info.num_subcores
    tpw = _NP // nw
    mesh = plsc.VectorSubcoreMesh(core_axis_name="c", subcore_axis_name="s")
    ncand = _K * _CH

    @functools.partial(
        pl.kernel, mesh=mesh,
        out_type=jax.ShapeDtypeStruct((_NP, _D), jnp.float32),
        compiler_params=pltpu.CompilerParams(needs_layout_passes=False),
        scratch_types=[
            pltpu.VMEM((tpw, _K), jnp.int32),
            pltpu.VMEM((tpw + 16,), jnp.float32),
            pltpu.VMEM((_K, _CH), jnp.float32),
            pltpu.VMEM((ncand + 16,), jnp.float32),
            pltpu.VMEM((ncand + 16,), jnp.int32),
            pltpu.VMEM((_K + 16,), jnp.float32),
            pltpu.VMEM((_K + 16,), jnp.int32),
            pltpu.VMEM((_K, _D), jnp.float32),
            pltpu.VMEM((1, _D), jnp.float32),
            pltpu.VMEM((1, _D), jnp.float32),
            pltpu.SemaphoreType.DMA,
            pltpu.SemaphoreType.DMA,
        ])
    def sc_k(acts_hbm, ids_hbm, th_hbm, wdec_hbm, bdec_hbm, out_hbm,
             ids_v, theta_v, chunks_v, cand_val, cand_idx, sel_val, sel_idx,
             rows_v, bdec_v, acc_v, sem, sem2):
        cc = lax.axis_index("c")
        ss = lax.axis_index("s")
        wid = ss * info.num_cores + cc
        base = wid * tpw
        pltpu.sync_copy(ids_hbm.at[pl.ds(base, tpw), :], ids_v)
        pltpu.sync_copy(th_hbm.at[pl.ds(base, tpw)],
                        theta_v.at[pl.ds(0, tpw)])
        pltpu.sync_copy(bdec_hbm, bdec_v)
        lane = lax.broadcasted_iota(jnp.int32, (16,), 0)

        def token_body(t, _):
            tok = base + t
            pltpu.async_copy(acts_hbm.at[ids_v.at[t]], chunks_v, sem).wait()
            th_v = jnp.full((16,), theta_v[pl.ds(t, 16)][0], jnp.float32)
            featbase0 = tok * _F
            ids_row = (ids_v[t, 0:16], ids_v[t, 16:32])
            cnt = jnp.int32(0)
            for j in range(_K):
                fj = ids_row[j // 16][j % 16] * _CH - featbase0
                for l in range(_CH // 16):
                    v = chunks_v[j, l * 16:(l + 1) * 16]
                    m = v >= th_v
                    iv = jnp.full((16,), fj + l * 16, jnp.int32) + lane
                    mi = jnp.where(m, 1, 0).astype(jnp.int32)
                    pos = plsc.cumsum(mi) - 1 + cnt
                    plsc.store_scatter(cand_val, [pos], v, mask=m)
                    plsc.store_scatter(cand_idx, [pos], iv, mask=m)
                    cnt = cnt + jnp.sum(mi)
            nv = (cnt + 15) // 16

            def count_ge(vec):
                def cbody(i, acc):
                    v = cand_val[pl.ds(i * 16, 16)]
                    valid = (lane + i * 16) < cnt
                    return acc + jnp.sum(
                        jnp.where(valid & (v >= vec), 1, 0).astype(jnp.int32))
                return lax.fori_loop(0, nv, cbody, jnp.int32(0))

            def sbody(_, lohi):
                lo, hi = lohi
                mid = lo + (hi - lo + 1) // 2
                midf = plsc.bitcast(jnp.full((16,), mid, jnp.int32),
                                    jnp.float32)
                big = count_ge(midf) >= _K
                return (jnp.where(big, mid, lo),
                        jnp.where(big, hi, mid - 1))

            lo, _hi = lax.fori_loop(
                0, 31, sbody, (jnp.int32(0), jnp.int32(0x7F800000)))
            tstar = plsc.bitcast(jnp.full((16,), lo, jnp.int32), jnp.float32)

            def gbody(i, acc):
                v = cand_val[pl.ds(i * 16, 16)]
                valid = (lane + i * 16) < cnt
                return acc + jnp.sum(
                    jnp.where(valid & (v > tstar), 1, 0).astype(jnp.int32))

            n_gt = lax.fori_loop(0, nv, gbody, jnp.int32(0))
            need = _K - n_gt

            def selbody(i, carry):
                scnt, eqs = carry
                v = cand_val[pl.ds(i * 16, 16)]
                iv = cand_idx[pl.ds(i * 16, 16)]
                valid = (lane + i * 16) < cnt
                m_gt = valid & (v > tstar)
                m_eq = valid & (v == tstar)
                meqi = jnp.where(m_eq, 1, 0).astype(jnp.int32)
                rank = plsc.cumsum(meqi)
                sel = m_gt | (m_eq & ((rank + eqs) <= need))
                seli = jnp.where(sel, 1, 0).astype(jnp.int32)
                pos = plsc.cumsum(seli) - 1 + scnt
                plsc.store_scatter(sel_val, [pos], v, mask=sel)
                plsc.store_scatter(sel_idx, [pos], iv, mask=sel)
                return (scnt + jnp.sum(seli), eqs + jnp.sum(meqi))

            lax.fori_loop(0, nv, selbody, (jnp.int32(0), jnp.int32(0)))

            pltpu.async_copy(wdec_hbm.at[sel_idx.at[pl.ds(0, _K)]],
                             rows_v, sem2).wait()
            sv_row = (sel_val[0:16], sel_val[16:32])
            splats = tuple(
                jnp.full((16,), sv_row[j // 16][j % 16], jnp.float32)
                for j in range(_K))

            def gloop(g, carry):
                acc = bdec_v[0, pl.ds(g * 16, 16)]
                for j in range(_K):
                    acc = acc + carry[j] * rows_v[j, pl.ds(g * 16, 16)]
                acc_v[0, pl.ds(g * 16, 16)] = acc
                return carry

            lax.fori_loop(0, _D // 16, gloop, splats)
            pltpu.sync_copy(acc_v, out_hbm.at[pl.ds(tok, 1), :])
            return jnp.int32(0)

        lax.fori_loop(0, tpw, token_body, jnp.int32(0))

    return sc_k(acts_flat, ids, theta, W_dec, b_dec2)


def kernel(x, W_enc, W_dec, b_dec):
    B, S, D = x.shape
    N = B * S
    xf = x.reshape(N, D)
    xf = jnp.concatenate(
        [xf, jnp.broadcast_to(xf[:1], (_NP - N, D))], axis=0)
    b2 = b_dec.reshape(1, D)

    acts3 = pl.pallas_call(
        _enc_body,
        grid=(_NP // _TB_MM, _F // _FB),
        in_specs=[
            pl.BlockSpec((_TB_MM, D), lambda t, f: (t, 0)),
            pl.BlockSpec((D, _FB), lambda t, f: (0, f)),
            pl.BlockSpec((1, D), lambda t, f: (0, 0)),
        ],
        out_specs=pl.BlockSpec((_TB_MM, _FB // _CH, _CH),
                               lambda t, f: (t, f, 0)),
        out_shape=jax.ShapeDtypeStruct((_NP, _NCH, _CH), jnp.float32),
    )(xf, W_enc, b2)

    cm = pl.pallas_call(
        _cmax_body,
        grid=(_NP // _TB_SEL,),
        in_specs=[pl.BlockSpec((_TB_SEL, _NCH, _CH), lambda t: (t, 0, 0))],
        out_specs=pl.BlockSpec((_TB_SEL, _NCH), lambda t: (t, 0)),
        out_shape=jax.ShapeDtypeStruct((_NP, _NCH), jnp.float32),
    )(acts3)
    acts_flat = acts3.reshape(_NP * _NCH, _CH)

    ids, theta = pl.pallas_call(
        _chunksel_body,
        grid=(_NP // _TB_SEL,),
        in_specs=[pl.BlockSpec((_TB_SEL, _NCH), lambda t: (t, 0))],
        out_specs=[
            pl.BlockSpec((_TB_SEL, _K), lambda t: (t, 0)),
            pl.BlockSpec((_TB_SEL, 1), lambda t: (t, 0)),
        ],
        out_shape=[
            jax.ShapeDtypeStruct((_NP, _K), jnp.int32),
            jax.ShapeDtypeStruct((_NP, 1), jnp.float32),
        ],
    )(cm)

    xhat = _sc_select_decode(acts_flat, ids, theta.reshape(_NP), W_dec, b2)
    return xhat[:N].reshape(B, S, D)
